# baseline (speedup 1.0000x reference)
import functools

import jax
import jax.numpy as jnp
from jax.experimental import pallas as pl
from jax.experimental.pallas import tpu as pltpu
from jax.experimental.pallas import tpu_sc as plsc

T, D, E, D_FF = 2048, 768, 64, 3072
TT = 32
TP = 4096
CH = 256
NTG = TP // TT
NW = 32
TOKW = T // NW


def _router_body(x_ref, wg_ref, logits_ref, pos_ref, w16_ref, base_ref, ntiles_ref):
    x = x_ref[...]
    logits = jnp.dot(x, wg_ref[...], preferred_element_type=jnp.float32)
    logits_ref[...] = logits

    lmax = jnp.max(logits, axis=1, keepdims=True)
    lane = jax.lax.broadcasted_iota(jnp.int32, (T, E), 1)
    eidx = jnp.min(jnp.where(logits >= lmax, lane, E), axis=1, keepdims=True)
    one_hot = (lane == eidx).astype(jnp.float32)
    denom = jnp.sum(jnp.exp(logits - lmax), axis=1, keepdims=True)
    w16_ref[...] = jnp.broadcast_to(1.0 / denom, (T, 128))

    li = jax.lax.broadcasted_iota(jnp.int32, (CH, CH), 0)
    lj = jax.lax.broadcasted_iota(jnp.int32, (CH, CH), 1)
    ltri = (lj < li).astype(jnp.float32)
    running = jnp.zeros((1, E), jnp.float32)
    rank_chunks = []
    for c in range(T // CH):
        chunk = one_hot[c * CH:(c + 1) * CH, :]
        within = jnp.dot(ltri, chunk, preferred_element_type=jnp.float32)
        rank_chunks.append(within + running)
        running = running + jnp.sum(chunk, axis=0, keepdims=True)
    ranks = jnp.concatenate(rank_chunks, axis=0)
    counts = running

    ntiles = jnp.floor((counts + (TT - 1)) / TT)
    padded = ntiles * TT
    ui = jax.lax.broadcasted_iota(jnp.int32, (E, E), 0)
    uj = jax.lax.broadcasted_iota(jnp.int32, (E, E), 1)
    upper = (ui < uj).astype(jnp.float32)
    base = jnp.sum(upper * jnp.reshape(padded, (E, 1)), axis=0, keepdims=True)

    pos = jnp.sum(one_hot * (ranks + base), axis=1, keepdims=True)
    pos_ref[...] = pos.astype(jnp.int32)
    base_ref[...] = base.astype(jnp.int32)
    ntiles_ref[...] = ntiles.astype(jnp.int32)


def _run_router(x, wg):
    return pl.pallas_call(
        _router_body,
        out_shape=[
            jax.ShapeDtypeStruct((T, E), jnp.float32),
            jax.ShapeDtypeStruct((T, 1), jnp.int32),
            jax.ShapeDtypeStruct((T, 128), jnp.float32),
            jax.ShapeDtypeStruct((1, E), jnp.int32),
            jax.ShapeDtypeStruct((1, E), jnp.int32),
        ],
    )(x, wg)


NB = 2


def _win_copy(win_hbm, win_buf, sin, e):
    return pltpu.make_async_copy(
        win_hbm.at[e], win_buf.at[e % NB], sin.at[e % NB]
    )


def _wout_copy(wout_hbm, wout_buf, sout, e):
    return pltpu.make_async_copy(
        wout_hbm.at[e], wout_buf.at[e % NB], sout.at[e % NB]
    )


def _moe_body(base_ref, ntiles_ref, xs_hbm, win_hbm, wout_hbm, out_hbm,
              win_buf, wout_buf, obuf, xbuf, sin, sout, so_sem, sx_sem):
    e = pl.program_id(0)
    base = base_ref[e]
    nt = ntiles_ref[e]
    gbase = base // TT
    gtot = (base_ref[E - 1] + ntiles_ref[E - 1] * TT) // TT

    @pl.when(e == 0)
    def _():
        _win_copy(win_hbm, win_buf, sin, 0).start()
        _wout_copy(wout_hbm, wout_buf, sout, 0).start()
        pltpu.make_async_copy(
            xs_hbm.at[pl.ds(0, TT)], xbuf.at[0], sx_sem.at[0]
        ).start()

    @pl.when(e + 1 < E)
    def _():
        _win_copy(win_hbm, win_buf, sin, e + 1).start()
        _wout_copy(wout_hbm, wout_buf, sout, e + 1).start()

    _win_copy(win_hbm, win_buf, sin, e).wait()
    _wout_copy(wout_hbm, wout_buf, sout, e).wait()

    win = win_buf[e % NB]
    wout = wout_buf[e % NB]

    def tile_body(i, carry):
        g = gbase + i
        oslot = jax.lax.rem(g, 2)

        @pl.when(g + 1 < gtot)
        def _():
            pltpu.make_async_copy(
                xs_hbm.at[pl.ds((g + 1) * TT, TT)], xbuf.at[jax.lax.rem(g + 1, 2)],
                sx_sem.at[jax.lax.rem(g + 1, 2)],
            ).start()

        @pl.when(g >= 2)
        def _():
            pltpu.make_async_copy(
                obuf.at[oslot], out_hbm.at[pl.ds((g - 2) * TT, TT)],
                so_sem.at[oslot],
            ).wait()

        start = pl.multiple_of(base + i * TT, TT)
        pltpu.make_async_copy(
            xs_hbm.at[pl.ds(start, TT)], xbuf.at[oslot], sx_sem.at[oslot]
        ).wait()
        xt = xbuf[oslot]
        h = jax.nn.gelu(
            jnp.dot(xt, win, preferred_element_type=jnp.float32),
            approximate=True,
        )
        obuf[oslot] = jnp.dot(h, wout, preferred_element_type=jnp.float32)
        pltpu.make_async_copy(
            obuf.at[oslot], out_hbm.at[pl.ds(start, TT)], so_sem.at[oslot]
        ).start()
        return carry

    jax.lax.fori_loop(0, nt, tile_body, 0)

    @pl.when(e == E - 1)
    def _():
        gtot = (base_ref[E - 1] + ntiles_ref[E - 1] * TT) // TT

        @pl.when(gtot >= 2)
        def _():
            g = gtot - 2
            pltpu.make_async_copy(
                obuf.at[jax.lax.rem(g, 2)], out_hbm.at[pl.ds(g * TT, TT)],
                so_sem.at[jax.lax.rem(g, 2)],
            ).wait()

        @pl.when(gtot >= 1)
        def _():
            g = gtot - 1
            pltpu.make_async_copy(
                obuf.at[jax.lax.rem(g, 2)], out_hbm.at[pl.ds(g * TT, TT)],
                so_sem.at[jax.lax.rem(g, 2)],
            ).wait()


def _run_moe(base, ntiles, xs, w_in, w_out):
    grid_spec = pltpu.PrefetchScalarGridSpec(
        num_scalar_prefetch=2,
        grid=(E,),
        in_specs=[
            pl.BlockSpec(memory_space=pltpu.MemorySpace.HBM),
            pl.BlockSpec(memory_space=pltpu.MemorySpace.HBM),
            pl.BlockSpec(memory_space=pltpu.MemorySpace.HBM),
        ],
        out_specs=pl.BlockSpec(memory_space=pltpu.MemorySpace.HBM),
        scratch_shapes=[
            pltpu.VMEM((NB, D, D_FF), jnp.float32),
            pltpu.VMEM((NB, D_FF, D), jnp.float32),
            pltpu.VMEM((2, TT, D), jnp.float32),
            pltpu.VMEM((2, TT, D), jnp.float32),
            pltpu.SemaphoreType.DMA((NB,)),
            pltpu.SemaphoreType.DMA((NB,)),
            pltpu.SemaphoreType.DMA((2,)),
            pltpu.SemaphoreType.DMA((2,)),
        ],
    )
    return pl.pallas_call(
        _moe_body,
        grid_spec=grid_spec,
        out_shape=jax.ShapeDtypeStruct((TP, D), jnp.float32),
        compiler_params=pltpu.CompilerParams(
            dimension_semantics=("arbitrary",),
            vmem_limit_bytes=110 * 1024 * 1024,
        ),
    )(base, ntiles, xs, w_in, w_out)


@functools.cache
def _sc_kernels():
    mesh = plsc.VectorSubcoreMesh(core_axis_name="c", subcore_axis_name="s")

    @functools.partial(
        pl.kernel,
        out_type=jax.ShapeDtypeStruct((TP, D), jnp.float32),
        mesh=mesh,
        scratch_types=[
            pltpu.VMEM((TOKW,), jnp.int32),
            pltpu.VMEM((TOKW, D), jnp.float32),
            pltpu.SemaphoreType.DMA,
        ],
    )
    def dispatch(x_hbm, pos_hbm, xs_hbm, idx_v, x_v, sem1):
        wid = jax.lax.axis_index("s") * 2 + jax.lax.axis_index("c")
        tok0 = wid * TOKW
        pltpu.sync_copy(pos_hbm.at[pl.ds(tok0, TOKW)], idx_v)
        pltpu.sync_copy(x_hbm.at[pl.ds(tok0, TOKW)], x_v)
        pltpu.async_copy(x_v, xs_hbm.at[idx_v], sem1).wait()

    @functools.partial(
        pl.kernel,
        out_type=jax.ShapeDtypeStruct((T, D), jnp.float32),
        mesh=mesh,
        scratch_types=[
            pltpu.VMEM((TOKW,), jnp.int32),
            pltpu.VMEM((TOKW, D), jnp.float32),
            pltpu.VMEM((TOKW, 128), jnp.float32),
            pltpu.SemaphoreType.DMA,
        ],
    )
    def collect(outs_hbm, pos_hbm, w128_hbm, out_hbm, idx_v, rows_v, w_v, sem):
        wid = jax.lax.axis_index("s") * 2 + jax.lax.axis_index("c")
        tok0 = wid * TOKW
        pltpu.sync_copy(pos_hbm.at[pl.ds(tok0, TOKW)], idx_v)
        pltpu.sync_copy(w128_hbm.at[pl.ds(tok0, TOKW)], w_v)
        pltpu.async_copy(outs_hbm.at[idx_v], rows_v, sem).wait()

        def row_body(r, carry):
            wr = w_v[r, 0:16]

            def col_body(c, carry2):
                sl = pl.ds(c * 16, 16)
                rows_v[r, sl] = rows_v[r, sl] * wr
                return carry2

            jax.lax.fori_loop(0, D // 16, col_body, 0, unroll=8)
            return carry

        jax.lax.fori_loop(0, TOKW, row_body, 0)
        pltpu.sync_copy(rows_v, out_hbm.at[pl.ds(tok0, TOKW)])

    return dispatch, collect


def kernel(hidden_states, W_gate, W_in, W_out):
    dispatch, collect = _sc_kernels()
    logits, pos2d, w128, base2d, ntiles2d = _run_router(hidden_states, W_gate)
    pos = pos2d.reshape(T)
    xs = dispatch(hidden_states, pos)
    outs = _run_moe(base2d.reshape(E), ntiles2d.reshape(E), xs, W_in, W_out)
    out = collect(outs, pos, w128)
    return out, logits


# speedup vs baseline: 1.1928x; 1.1928x over previous
import functools

import jax
import jax.numpy as jnp
from jax.experimental import pallas as pl
from jax.experimental.pallas import tpu as pltpu
from jax.experimental.pallas import tpu_sc as plsc

T, D, E, D_FF = 2048, 768, 64, 3072
TT = 32
TP = 4096
CH = 256
NTG = TP // TT
NW = 32
TOKW = T // NW


def _router_body(x_ref, wg_ref, logits_ref, pos_ref, w16_ref, base_ref, ntiles_ref):
    x = x_ref[...]
    logits = jnp.dot(x, wg_ref[...], preferred_element_type=jnp.float32)
    logits_ref[...] = logits

    lmax = jnp.max(logits, axis=1, keepdims=True)
    lane = jax.lax.broadcasted_iota(jnp.int32, (T, E), 1)
    eidx = jnp.min(jnp.where(logits >= lmax, lane, E), axis=1, keepdims=True)
    one_hot = (lane == eidx).astype(jnp.float32)
    denom = jnp.sum(jnp.exp(logits - lmax), axis=1, keepdims=True)
    w16_ref[...] = jnp.broadcast_to(1.0 / denom, (T, 128))

    li = jax.lax.broadcasted_iota(jnp.int32, (CH, CH), 0)
    lj = jax.lax.broadcasted_iota(jnp.int32, (CH, CH), 1)
    ltri = (lj < li).astype(jnp.float32)
    running = jnp.zeros((1, E), jnp.float32)
    rank_chunks = []
    for c in range(T // CH):
        chunk = one_hot[c * CH:(c + 1) * CH, :]
        within = jnp.dot(ltri, chunk, preferred_element_type=jnp.float32)
        rank_chunks.append(within + running)
        running = running + jnp.sum(chunk, axis=0, keepdims=True)
    ranks = jnp.concatenate(rank_chunks, axis=0)
    counts = running

    ntiles = jnp.floor((counts + (TT - 1)) / TT)
    padded = ntiles * TT
    ui = jax.lax.broadcasted_iota(jnp.int32, (E, E), 0)
    uj = jax.lax.broadcasted_iota(jnp.int32, (E, E), 1)
    upper = (ui < uj).astype(jnp.float32)
    base = jnp.sum(upper * jnp.reshape(padded, (E, 1)), axis=0, keepdims=True)

    pos = jnp.sum(one_hot * (ranks + base), axis=1, keepdims=True)
    pos_ref[...] = pos.astype(jnp.int32)
    base_ref[...] = base.astype(jnp.int32)
    ntiles_ref[...] = ntiles.astype(jnp.int32)


def _run_router(x, wg):
    return pl.pallas_call(
        _router_body,
        out_shape=[
            jax.ShapeDtypeStruct((T, E), jnp.float32),
            jax.ShapeDtypeStruct((T, 1), jnp.int32),
            jax.ShapeDtypeStruct((T, 128), jnp.float32),
            jax.ShapeDtypeStruct((1, E), jnp.int32),
            jax.ShapeDtypeStruct((1, E), jnp.int32),
        ],
    )(x, wg)


NF = 2
FF = D_FF // NF


NB = 3
SN = E * NF


DH = D // 2
FH = FF // 2


def _win_copies(win_hbm, win_buf, sin, step):
    e2 = step // NF
    f2 = step % NF
    slot = step % NB
    return [
        pltpu.make_async_copy(
            win_hbm.at[e2, pl.ds(h * DH, DH), pl.ds(f2 * FF, FF)],
            win_buf.at[slot, pl.ds(h * DH, DH)],
            sin.at[slot],
        )
        for h in (0, 1)
    ]


def _wout_copies(wout_hbm, wout_buf, sout, step):
    e2 = step // NF
    f2 = step % NF
    slot = step % NB
    return [
        pltpu.make_async_copy(
            wout_hbm.at[e2, pl.ds(f2 * FF + h * FH, FH), :],
            wout_buf.at[slot, pl.ds(h * FH, FH)],
            sout.at[slot],
        )
        for h in (0, 1)
    ]


def _start_step(win_hbm, wout_hbm, win_buf, wout_buf, sin, sout, step):
    for cp in _win_copies(win_hbm, win_buf, sin, step):
        cp.start()
    for cp in _wout_copies(wout_hbm, wout_buf, sout, step):
        cp.start()


def _moe_body(base_ref, ntiles_ref, xs_ref, win_hbm, wout_hbm, out_ref,
              win_buf, wout_buf, sin, sout):
    e = pl.program_id(0)
    f = pl.program_id(1)
    s = e * NF + f
    base = base_ref[e]
    nt = ntiles_ref[e]

    @pl.when(s == 0)
    def _():
        _start_step(win_hbm, wout_hbm, win_buf, wout_buf, sin, sout, 0)
        _start_step(win_hbm, wout_hbm, win_buf, wout_buf, sin, sout, 1)

    @pl.when(s + 2 < SN)
    def _():
        _start_step(win_hbm, wout_hbm, win_buf, wout_buf, sin, sout, s + 2)

    for cp in _win_copies(win_hbm, win_buf, sin, s):
        cp.wait()
    for cp in _wout_copies(wout_hbm, wout_buf, sout, s):
        cp.wait()

    slot = s % NB
    win = win_buf[slot]
    wout = wout_buf[slot]

    def tile_body(i, carry):
        start = pl.multiple_of(base + i * TT, TT)
        xt = xs_ref[pl.ds(start, TT), :]
        h = jax.nn.gelu(
            jnp.dot(xt, win, preferred_element_type=jnp.float32),
            approximate=True,
        )
        o = jnp.dot(h, wout, preferred_element_type=jnp.float32)

        @pl.when(f == 0)
        def _():
            out_ref[pl.ds(start, TT), :] = o

        @pl.when(f != 0)
        def _():
            out_ref[pl.ds(start, TT), :] += o

        return carry

    jax.lax.fori_loop(0, nt, tile_body, 0)


def _run_moe(base, ntiles, xs, w_in, w_out):
    grid_spec = pltpu.PrefetchScalarGridSpec(
        num_scalar_prefetch=2,
        grid=(E, NF),
        in_specs=[
            pl.BlockSpec((TP, D), lambda e, f, *_: (0, 0)),
            pl.BlockSpec(memory_space=pltpu.MemorySpace.HBM),
            pl.BlockSpec(memory_space=pltpu.MemorySpace.HBM),
        ],
        out_specs=pl.BlockSpec((TP, D), lambda e, f, *_: (0, 0)),
        scratch_shapes=[
            pltpu.VMEM((NB, D, FF), jnp.float32),
            pltpu.VMEM((NB, FF, D), jnp.float32),
            pltpu.SemaphoreType.DMA((NB,)),
            pltpu.SemaphoreType.DMA((NB,)),
        ],
    )
    return pl.pallas_call(
        _moe_body,
        grid_spec=grid_spec,
        out_shape=jax.ShapeDtypeStruct((TP, D), jnp.float32),
        compiler_params=pltpu.CompilerParams(
            dimension_semantics=("arbitrary", "arbitrary"),
            vmem_limit_bytes=110 * 1024 * 1024,
        ),
    )(base, ntiles, xs, w_in, w_out)


@functools.cache
def _sc_kernels():
    mesh = plsc.VectorSubcoreMesh(core_axis_name="c", subcore_axis_name="s")

    @functools.partial(
        pl.kernel,
        out_type=jax.ShapeDtypeStruct((TP, D), jnp.float32),
        mesh=mesh,
        scratch_types=[
            pltpu.VMEM((TOKW,), jnp.int32),
            pltpu.VMEM((TOKW, D), jnp.float32),
            pltpu.SemaphoreType.DMA,
        ],
    )
    def dispatch(x_hbm, pos_hbm, xs_hbm, idx_v, x_v, sem1):
        wid = jax.lax.axis_index("s") * 2 + jax.lax.axis_index("c")
        tok0 = wid * TOKW
        pltpu.sync_copy(pos_hbm.at[pl.ds(tok0, TOKW)], idx_v)
        pltpu.sync_copy(x_hbm.at[pl.ds(tok0, TOKW)], x_v)
        pltpu.async_copy(x_v, xs_hbm.at[idx_v], sem1).wait()

    @functools.partial(
        pl.kernel,
        out_type=jax.ShapeDtypeStruct((T, D), jnp.float32),
        mesh=mesh,
        scratch_types=[
            pltpu.VMEM((TOKW,), jnp.int32),
            pltpu.VMEM((TOKW, D), jnp.float32),
            pltpu.VMEM((TOKW, 128), jnp.float32),
            pltpu.SemaphoreType.DMA,
        ],
    )
    def collect(outs_hbm, pos_hbm, w128_hbm, out_hbm, idx_v, rows_v, w_v, sem):
        wid = jax.lax.axis_index("s") * 2 + jax.lax.axis_index("c")
        tok0 = wid * TOKW
        pltpu.sync_copy(pos_hbm.at[pl.ds(tok0, TOKW)], idx_v)
        pltpu.sync_copy(w128_hbm.at[pl.ds(tok0, TOKW)], w_v)
        pltpu.async_copy(outs_hbm.at[idx_v], rows_v, sem).wait()

        def row_body(r, carry):
            wr = w_v[r, 0:16]

            def col_body(c, carry2):
                sl = pl.ds(c * 16, 16)
                rows_v[r, sl] = rows_v[r, sl] * wr
                return carry2

            jax.lax.fori_loop(0, D // 16, col_body, 0, unroll=8)
            return carry

        jax.lax.fori_loop(0, TOKW, row_body, 0)
        pltpu.sync_copy(rows_v, out_hbm.at[pl.ds(tok0, TOKW)])

    return dispatch, collect


def kernel(hidden_states, W_gate, W_in, W_out):
    dispatch, collect = _sc_kernels()
    logits, pos2d, w128, base2d, ntiles2d = _run_router(hidden_states, W_gate)
    pos = pos2d.reshape(T)
    xs = dispatch(hidden_states, pos)
    outs = _run_moe(base2d.reshape(E), ntiles2d.reshape(E), xs, W_in, W_out)
    out = collect(outs, pos, w128)
    return out, logits


# speedup vs baseline: 1.1935x; 1.0006x over previous
import functools

import jax
import jax.numpy as jnp
from jax.experimental import pallas as pl
from jax.experimental.pallas import tpu as pltpu
from jax.experimental.pallas import tpu_sc as plsc

T, D, E, D_FF = 2048, 768, 64, 3072
TT = 32
TP = 4096
CH = 256
NTG = TP // TT
NW = 32
TOKW = T // NW


def _router_body(x_ref, wg_ref, logits_ref, pos_ref, w16_ref, base_ref, ntiles_ref):
    x = x_ref[...]
    logits = jnp.dot(x, wg_ref[...], preferred_element_type=jnp.float32)
    logits_ref[...] = logits

    lmax = jnp.max(logits, axis=1, keepdims=True)
    lane = jax.lax.broadcasted_iota(jnp.int32, (T, E), 1)
    eidx = jnp.min(jnp.where(logits >= lmax, lane, E), axis=1, keepdims=True)
    one_hot = (lane == eidx).astype(jnp.float32)
    denom = jnp.sum(jnp.exp(logits - lmax), axis=1, keepdims=True)
    w16_ref[...] = jnp.broadcast_to(1.0 / denom, (T, 128))

    li = jax.lax.broadcasted_iota(jnp.int32, (CH, CH), 0)
    lj = jax.lax.broadcasted_iota(jnp.int32, (CH, CH), 1)
    ltri = (lj < li).astype(jnp.float32)
    running = jnp.zeros((1, E), jnp.float32)
    rank_chunks = []
    for c in range(T // CH):
        chunk = one_hot[c * CH:(c + 1) * CH, :]
        within = jnp.dot(ltri, chunk, preferred_element_type=jnp.float32)
        rank_chunks.append(within + running)
        running = running + jnp.sum(chunk, axis=0, keepdims=True)
    ranks = jnp.concatenate(rank_chunks, axis=0)
    counts = running

    ntiles = jnp.floor((counts + (TT - 1)) / TT)
    padded = ntiles * TT
    ui = jax.lax.broadcasted_iota(jnp.int32, (E, E), 0)
    uj = jax.lax.broadcasted_iota(jnp.int32, (E, E), 1)
    upper = (ui < uj).astype(jnp.float32)
    base = jnp.sum(upper * jnp.reshape(padded, (E, 1)), axis=0, keepdims=True)

    pos = jnp.sum(one_hot * (ranks + base), axis=1, keepdims=True)
    pos_ref[...] = pos.astype(jnp.int32)
    base_ref[...] = base.astype(jnp.int32)
    ntiles_ref[...] = ntiles.astype(jnp.int32)


def _run_router(x, wg):
    return pl.pallas_call(
        _router_body,
        out_shape=[
            jax.ShapeDtypeStruct((T, E), jnp.float32),
            jax.ShapeDtypeStruct((T, 1), jnp.int32),
            jax.ShapeDtypeStruct((T, 128), jnp.float32),
            jax.ShapeDtypeStruct((1, E), jnp.int32),
            jax.ShapeDtypeStruct((1, E), jnp.int32),
        ],
    )(x, wg)


NF = 2
FF = D_FF // NF


NB = 3
SN = E * NF


NSPLIT = 4
DH = D // NSPLIT
FH = FF // NSPLIT


def _win_copies(win_hbm, win_buf, sin, step):
    e2 = step // NF
    f2 = step % NF
    slot = step % NB
    return [
        pltpu.make_async_copy(
            win_hbm.at[e2, pl.ds(h * DH, DH), pl.ds(f2 * FF, FF)],
            win_buf.at[slot, pl.ds(h * DH, DH)],
            sin.at[slot],
        )
        for h in range(NSPLIT)
    ]


def _wout_copies(wout_hbm, wout_buf, sout, step):
    e2 = step // NF
    f2 = step % NF
    slot = step % NB
    return [
        pltpu.make_async_copy(
            wout_hbm.at[e2, pl.ds(f2 * FF + h * FH, FH), :],
            wout_buf.at[slot, pl.ds(h * FH, FH)],
            sout.at[slot],
        )
        for h in range(NSPLIT)
    ]


def _start_step(win_hbm, wout_hbm, win_buf, wout_buf, sin, sout, step):
    for cp in _win_copies(win_hbm, win_buf, sin, step):
        cp.start()
    for cp in _wout_copies(wout_hbm, wout_buf, sout, step):
        cp.start()


def _moe_body(base_ref, ntiles_ref, xs_ref, win_hbm, wout_hbm, out_ref,
              win_buf, wout_buf, sin, sout):
    e = pl.program_id(0)
    f = pl.program_id(1)
    s = e * NF + f
    base = base_ref[e]
    nt = ntiles_ref[e]

    @pl.when(s == 0)
    def _():
        _start_step(win_hbm, wout_hbm, win_buf, wout_buf, sin, sout, 0)
        _start_step(win_hbm, wout_hbm, win_buf, wout_buf, sin, sout, 1)

    @pl.when(s + 2 < SN)
    def _():
        _start_step(win_hbm, wout_hbm, win_buf, wout_buf, sin, sout, s + 2)

    for cp in _win_copies(win_hbm, win_buf, sin, s):
        cp.wait()
    for cp in _wout_copies(wout_hbm, wout_buf, sout, s):
        cp.wait()

    slot = s % NB
    win = win_buf[slot]
    wout = wout_buf[slot]

    def tile_body(i, carry):
        start = pl.multiple_of(base + i * TT, TT)
        xt = xs_ref[pl.ds(start, TT), :]
        h = jax.nn.gelu(
            jnp.dot(xt, win, preferred_element_type=jnp.float32),
            approximate=True,
        )
        o = jnp.dot(h, wout, preferred_element_type=jnp.float32)

        @pl.when(f == 0)
        def _():
            out_ref[pl.ds(start, TT), :] = o

        @pl.when(f != 0)
        def _():
            out_ref[pl.ds(start, TT), :] += o

        return carry

    jax.lax.fori_loop(0, nt, tile_body, 0)


def _run_moe(base, ntiles, xs, w_in, w_out):
    grid_spec = pltpu.PrefetchScalarGridSpec(
        num_scalar_prefetch=2,
        grid=(E, NF),
        in_specs=[
            pl.BlockSpec((TP, D), lambda e, f, *_: (0, 0)),
            pl.BlockSpec(memory_space=pltpu.MemorySpace.HBM),
            pl.BlockSpec(memory_space=pltpu.MemorySpace.HBM),
        ],
        out_specs=pl.BlockSpec((TP, D), lambda e, f, *_: (0, 0)),
        scratch_shapes=[
            pltpu.VMEM((NB, D, FF), jnp.float32),
            pltpu.VMEM((NB, FF, D), jnp.float32),
            pltpu.SemaphoreType.DMA((NB,)),
            pltpu.SemaphoreType.DMA((NB,)),
        ],
    )
    return pl.pallas_call(
        _moe_body,
        grid_spec=grid_spec,
        out_shape=jax.ShapeDtypeStruct((TP, D), jnp.float32),
        compiler_params=pltpu.CompilerParams(
            dimension_semantics=("arbitrary", "arbitrary"),
            vmem_limit_bytes=110 * 1024 * 1024,
        ),
    )(base, ntiles, xs, w_in, w_out)


@functools.cache
def _sc_kernels():
    mesh = plsc.VectorSubcoreMesh(core_axis_name="c", subcore_axis_name="s")

    @functools.partial(
        pl.kernel,
        out_type=jax.ShapeDtypeStruct((TP, D), jnp.float32),
        mesh=mesh,
        scratch_types=[
            pltpu.VMEM((TOKW,), jnp.int32),
            pltpu.VMEM((TOKW, D), jnp.float32),
            pltpu.SemaphoreType.DMA,
        ],
    )
    def dispatch(x_hbm, pos_hbm, xs_hbm, idx_v, x_v, sem1):
        wid = jax.lax.axis_index("s") * 2 + jax.lax.axis_index("c")
        tok0 = wid * TOKW
        pltpu.sync_copy(pos_hbm.at[pl.ds(tok0, TOKW)], idx_v)
        pltpu.sync_copy(x_hbm.at[pl.ds(tok0, TOKW)], x_v)
        pltpu.async_copy(x_v, xs_hbm.at[idx_v], sem1).wait()

    @functools.partial(
        pl.kernel,
        out_type=jax.ShapeDtypeStruct((T, D), jnp.float32),
        mesh=mesh,
        scratch_types=[
            pltpu.VMEM((TOKW,), jnp.int32),
            pltpu.VMEM((TOKW, D), jnp.float32),
            pltpu.VMEM((TOKW, 128), jnp.float32),
            pltpu.SemaphoreType.DMA,
        ],
    )
    def collect(outs_hbm, pos_hbm, w128_hbm, out_hbm, idx_v, rows_v, w_v, sem):
        wid = jax.lax.axis_index("s") * 2 + jax.lax.axis_index("c")
        tok0 = wid * TOKW
        pltpu.sync_copy(pos_hbm.at[pl.ds(tok0, TOKW)], idx_v)
        pltpu.sync_copy(w128_hbm.at[pl.ds(tok0, TOKW)], w_v)
        pltpu.async_copy(outs_hbm.at[idx_v], rows_v, sem).wait()

        def row_body(r, carry):
            wr = w_v[r, 0:16]

            def col_body(c, carry2):
                sl = pl.ds(c * 16, 16)
                rows_v[r, sl] = rows_v[r, sl] * wr
                return carry2

            jax.lax.fori_loop(0, D // 16, col_body, 0, unroll=8)
            return carry

        jax.lax.fori_loop(0, TOKW, row_body, 0)
        pltpu.sync_copy(rows_v, out_hbm.at[pl.ds(tok0, TOKW)])

    return dispatch, collect


def kernel(hidden_states, W_gate, W_in, W_out):
    dispatch, collect = _sc_kernels()
    logits, pos2d, w128, base2d, ntiles2d = _run_router(hidden_states, W_gate)
    pos = pos2d.reshape(T)
    xs = dispatch(hidden_states, pos)
    outs = _run_moe(base2d.reshape(E), ntiles2d.reshape(E), xs, W_in, W_out)
    out = collect(outs, pos, w128)
    return out, logits
